# Initial kernel scaffold; baseline (speedup 1.0000x reference)
import functools

import jax
import jax.numpy as jnp
from jax import lax
from jax.experimental import pallas as pl
from jax.experimental.pallas import tpu as pltpu
from jax.experimental.pallas import tpu_sc as plsc

N = 50000
NFEAT = 128
NHID = 64
C = 47
CP = 48
K = 10

NC = 2
NS = 16

RW = 112
NP = 50176
HALF = NP // 2
PT_ROWS = HALF // NS
NRW = PT_ROWS // RW
TRASH = NS * 16
NROWS = HALF + TRASH

B = 128
KG = 4

_MESH = plsc.VectorSubcoreMesh(
    core_axis_name="c", subcore_axis_name="s", num_cores=NC, num_subcores=NS)


def _f32(shape):
    return jax.ShapeDtypeStruct(shape, jnp.float32)


def _make_hist(epad):
    chunks = epad // B
    per_sc = chunks // NC
    per_tile = per_sc // NS
    groups = per_tile // KG
    zslice = NP // NS

    @functools.partial(
        pl.kernel,
        out_type=_f32((NC, NP)),
        mesh=_MESH,
        scratch_types=[
            pltpu.VMEM_SHARED((NP,), jnp.float32),
            pltpu.VMEM((zslice,), jnp.float32),
            pltpu.VMEM((B,), jnp.float32),
            [pltpu.VMEM((B,), jnp.int32) for _ in range(KG)],
            pltpu.SemaphoreType.DMA,
            pltpu.SemaphoreType.DMA,
        ],
    )
    def hist(col_hbm, out_hbm, deg_sh, zbuf, onesb, colb, sem_l, sem_s):
        cid = lax.axis_index("c")
        tid = lax.axis_index("s")

        @pl.loop(0, zslice // 16)
        def _zero(i):
            zbuf[pl.ds(i * 16, 16)] = jnp.zeros((16,), jnp.float32)

        for s in range(B // 16):
            onesb[pl.ds(s * 16, 16)] = jnp.ones((16,), jnp.float32)
        pltpu.sync_copy(zbuf, deg_sh.at[pl.ds(tid * zslice, zslice)])
        plsc.subcore_barrier()

        base = cid * per_sc + tid * per_tile

        @pl.loop(0, groups)
        def _grp(gi):
            c0 = base + gi * KG
            dls = [
                pltpu.async_copy(
                    col_hbm.at[pl.ds((c0 + b) * B, B)], colb[b], sem_l)
                for b in range(KG)
            ]
            for d in dls:
                d.wait()
            dss = [
                pltpu.async_copy(onesb, deg_sh.at[colb[b]], sem_s, add=True)
                for b in range(KG)
            ]
            for d in dss:
                d.wait()

        plsc.subcore_barrier()
        pltpu.sync_copy(deg_sh.at[pl.ds(tid * zslice, zslice)],
                        out_hbm.at[cid, pl.ds(tid * zslice, zslice)])

    return hist


def _tc_body(x_ref, w1_ref, b1_ref, w2_ref, b2_ref, degs_ref, t0_ref,
             g0_ref, hid_ref, dinv_ref):
    xb = x_ref[...]
    h1 = jnp.maximum(
        jnp.dot(xb, w1_ref[...], preferred_element_type=jnp.float32)
        + b1_ref[...], 0.0)
    h = (jnp.dot(h1, w2_ref[...], preferred_element_type=jnp.float32)
         + b2_ref[...])
    deg = degs_ref[:, 0] + degs_ref[:, 1] + 1.0
    dinv = lax.rsqrt(deg)
    g0_ref[...] = h * dinv[:, None]
    hid_ref[...] = h * t0_ref[0, 0]
    dinv_ref[...] = dinv[:, None]


def _tc_mlp(x_p, W1, b1r, W2p, b2r, degs_t, t0):
    nb = NP // 128
    return pl.pallas_call(
        _tc_body,
        grid=(nb,),
        in_specs=[
            pl.BlockSpec((128, NFEAT), lambda i: (i, 0)),
            pl.BlockSpec((NFEAT, NHID), lambda i: (0, 0)),
            pl.BlockSpec((1, NHID), lambda i: (0, 0)),
            pl.BlockSpec((NHID, CP), lambda i: (0, 0)),
            pl.BlockSpec((1, CP), lambda i: (0, 0)),
            pl.BlockSpec((128, NC), lambda i: (i, 0)),
            pl.BlockSpec((1, 1), lambda i: (0, 0)),
        ],
        out_specs=[
            pl.BlockSpec((128, CP), lambda i: (i, 0)),
            pl.BlockSpec((128, CP), lambda i: (i, 0)),
            pl.BlockSpec((128, 1), lambda i: (i, 0)),
        ],
        out_shape=[_f32((NP, CP)), _f32((NP, CP)), _f32((NP, 1))],
    )(x_p, W1, b1r, W2p, b2r, degs_t, t0)


def _make_round(epad):
    chunks = epad // B
    per_tile = chunks // NS
    groups = per_tile // KG

    @functools.partial(
        pl.kernel,
        out_type=(_f32((NP, CP)), _f32((NP, CP))),
        mesh=_MESH,
        scratch_types=[
            pltpu.VMEM_SHARED((NROWS, CP), jnp.float32),
            [pltpu.VMEM((B,), jnp.int32) for _ in range(KG)],
            [pltpu.VMEM((B,), jnp.int32) for _ in range(KG)],
            [pltpu.VMEM((B,), jnp.int32) for _ in range(KG)],
            [pltpu.VMEM((B, CP), jnp.float32) for _ in range(KG)],
            pltpu.VMEM((RW, CP), jnp.float32),
            pltpu.VMEM((RW, CP), jnp.float32),
            pltpu.VMEM((RW, CP), jnp.float32),
            pltpu.VMEM((RW, CP), jnp.float32),
            pltpu.VMEM((RW,), jnp.float32),
            pltpu.VMEM((16,), jnp.float32),
            pltpu.SemaphoreType.DMA,
            pltpu.SemaphoreType.DMA,
            pltpu.SemaphoreType.DMA,
        ],
    )
    def rnd(row_hbm, col_hbm, gin_hbm, hin_hbm, dinv_hbm, tk_hbm,
            gout_hbm, hout_hbm,
            acc_sh, rowb, colb, clb, gbuf, zbuf, abuf, obuf, hbuf,
            dbuf, tkb, sem_l, sem_g, sem_s):
        cid = lax.axis_index("c")
        tid = lax.axis_index("s")
        sc_base = cid * HALF
        trash0 = HALF + tid * 16

        pltpu.sync_copy(tk_hbm, tkb)

        @pl.loop(0, RW)
        def _z(i):
            for j in range(CP // 16):
                zbuf[i, pl.ds(j * 16, 16)] = jnp.zeros((16,), jnp.float32)

        @pl.loop(0, NRW)
        def _za(i):
            pltpu.sync_copy(
                zbuf, acc_sh.at[pl.ds(tid * PT_ROWS + i * RW, RW)])

        plsc.subcore_barrier()

        base = tid * per_tile
        iota16 = lax.iota(jnp.int32, 16)

        @pl.loop(0, groups)
        def _grp(gi):
            c0 = base + gi * KG
            dls = []
            for b in range(KG):
                off = (c0 + b) * B
                dls.append(pltpu.async_copy(
                    row_hbm.at[pl.ds(off, B)], rowb[b], sem_l))
                dls.append(pltpu.async_copy(
                    col_hbm.at[pl.ds(off, B)], colb[b], sem_l))
            for d in dls:
                d.wait()
            dgs = []
            for b in range(KG):
                for s in range(B // 16):
                    cv = colb[b][pl.ds(s * 16, 16)] - sc_base
                    oob = (cv < 0) | (cv >= HALF)
                    clb[b][pl.ds(s * 16, 16)] = jnp.where(
                        oob, trash0 + iota16, cv)
                dgs.append(pltpu.async_copy(
                    gin_hbm.at[rowb[b]], gbuf[b], sem_g))
            for d in dgs:
                d.wait()
            dss = [
                pltpu.async_copy(gbuf[b], acc_sh.at[clb[b]], sem_s, add=True)
                for b in range(KG)
            ]
            for d in dss:
                d.wait()

        plsc.subcore_barrier()

        tkv = tkb[...]

        @pl.loop(0, NRW)
        def _dense(ci):
            lrow = tid * PT_ROWS + ci * RW
            grow = sc_base + lrow
            pltpu.sync_copy(acc_sh.at[pl.ds(lrow, RW)], abuf)
            pltpu.sync_copy(gin_hbm.at[pl.ds(grow, RW)], obuf)
            pltpu.sync_copy(hin_hbm.at[pl.ds(grow, RW)], hbuf)
            pltpu.sync_copy(dinv_hbm.at[pl.ds(grow, RW)], dbuf)

            @pl.loop(0, RW)
            def _row(r):
                dv = plsc.load_gather(dbuf, [jnp.full((16,), r, jnp.int32)])
                dsq = dv * dv
                ck = tkv * dv
                for j in range(CP // 16):
                    sl = pl.ds(j * 16, 16)
                    t = abuf[r, sl] + obuf[r, sl]
                    abuf[r, sl] = dsq * t
                    hbuf[r, sl] = hbuf[r, sl] + ck * t

            pltpu.sync_copy(abuf, gout_hbm.at[pl.ds(grow, RW)])
            pltpu.sync_copy(hbuf, hout_hbm.at[pl.ds(grow, RW)])

    return rnd


def kernel(x, edge_index, W1, b1, W2, b2, temp):
    E = edge_index.shape[1]
    epad = ((E + 16383) // 16384) * 16384
    npad_e = epad - E

    row = edge_index[0]
    col = edge_index[1]
    pad_idx = (N + (jnp.arange(npad_e, dtype=jnp.int32) % (NP - N))).astype(
        jnp.int32)
    row_p = jnp.concatenate([row, pad_idx])
    col_p = jnp.concatenate([col, pad_idx])

    x_p = jnp.zeros((NP, NFEAT), jnp.float32).at[:N].set(x)
    W2p = jnp.zeros((NHID, CP), jnp.float32).at[:, :C].set(W2)
    b2r = jnp.zeros((1, CP), jnp.float32).at[0, :C].set(b2)
    b1r = b1.reshape(1, NHID)
    t0 = temp[0].reshape(1, 1)

    degs = _make_hist(epad)(col_p)
    degs_t = degs.T

    g, hid, dinv2d = _tc_mlp(x_p, W1, b1r, W2p, b2r, degs_t, t0)
    dinv = dinv2d.reshape(NP)

    rnd = _make_round(epad)
    for k in range(K):
        tkv = jnp.broadcast_to(temp[k + 1], (16,)).astype(jnp.float32)
        g, hid = rnd(row_p, col_p, g, hid, dinv, tkv)

    return hid[:N, :C]


# speedup vs baseline: 14.8825x; 14.8825x over previous
import functools

import jax
import jax.numpy as jnp
from jax import lax
from jax.experimental import pallas as pl
from jax.experimental.pallas import tpu as pltpu
from jax.experimental.pallas import tpu_sc as plsc

N = 50000
NFEAT = 128
NHID = 64
C = 47
CP = 48
K = 10

NC = 2
NS = 16

RW = 112
NP = 50176
HALF = NP // 2
PT_ROWS = HALF // NS
NRW = PT_ROWS // RW
TRASH = NS * 16
NROWS = HALF + TRASH

B = 128
KG = 4

_MESH = plsc.VectorSubcoreMesh(
    core_axis_name="c", subcore_axis_name="s", num_cores=NC, num_subcores=NS)


def _f32(shape):
    return jax.ShapeDtypeStruct(shape, jnp.float32)


def _make_hist(epad):
    chunks = epad // B
    per_sc = chunks // NC
    per_tile = per_sc // NS
    groups = per_tile // KG
    zslice = NP // NS

    @functools.partial(
        pl.kernel,
        out_type=_f32((NC * NP,)),
        mesh=_MESH,
        scratch_types=[
            pltpu.VMEM_SHARED((NP,), jnp.float32),
            pltpu.VMEM((zslice,), jnp.float32),
            pltpu.VMEM((B,), jnp.float32),
            [pltpu.VMEM((B,), jnp.int32) for _ in range(KG)],
            pltpu.SemaphoreType.DMA,
            pltpu.SemaphoreType.DMA,
        ],
    )
    def hist(col_hbm, out_hbm, deg_sh, zbuf, onesb, colb, sem_l, sem_s):
        cid = lax.axis_index("c")
        tid = lax.axis_index("s")

        @pl.loop(0, zslice // 16)
        def _zero(i):
            zbuf[pl.ds(i * 16, 16)] = jnp.zeros((16,), jnp.float32)

        for s in range(B // 16):
            onesb[pl.ds(s * 16, 16)] = jnp.ones((16,), jnp.float32)
        pltpu.sync_copy(zbuf, deg_sh.at[pl.ds(tid * zslice, zslice)])
        plsc.subcore_barrier()

        base = cid * per_sc + tid * per_tile

        @pl.loop(0, groups)
        def _grp(gi):
            c0 = base + gi * KG
            dls = [
                pltpu.async_copy(
                    col_hbm.at[pl.ds((c0 + b) * B, B)], colb[b], sem_l)
                for b in range(KG)
            ]
            for d in dls:
                d.wait()
            dss = [
                pltpu.async_copy(onesb, deg_sh.at[colb[b]], sem_s, add=True)
                for b in range(KG)
            ]
            for d in dss:
                d.wait()

        plsc.subcore_barrier()
        pltpu.sync_copy(deg_sh.at[pl.ds(tid * zslice, zslice)], zbuf)
        pltpu.sync_copy(zbuf,
                        out_hbm.at[pl.ds(cid * NP + tid * zslice, zslice)])

    return hist


def _tc_body(x_ref, w1_ref, b1_ref, w2_ref, b2_ref, degs_ref, t0_ref,
             g0_ref, hid_ref, dinv_ref):
    xb = x_ref[...]
    h1 = jnp.maximum(
        jnp.dot(xb, w1_ref[...], preferred_element_type=jnp.float32)
        + b1_ref[...], 0.0)
    h = (jnp.dot(h1, w2_ref[...], preferred_element_type=jnp.float32)
         + b2_ref[...])
    deg = degs_ref[:, 0] + degs_ref[:, 1] + 1.0
    dinv = lax.rsqrt(deg)
    g0_ref[...] = h * dinv[:, None]
    hid_ref[...] = h * t0_ref[0, 0]
    dinv_ref[...] = jnp.broadcast_to(dinv[:, None], (128, 16))


def _tc_mlp(x_p, W1, b1r, W2p, b2r, degs_t, t0):
    nb = NP // 128
    return pl.pallas_call(
        _tc_body,
        grid=(nb,),
        in_specs=[
            pl.BlockSpec((128, NFEAT), lambda i: (i, 0)),
            pl.BlockSpec((NFEAT, NHID), lambda i: (0, 0)),
            pl.BlockSpec((1, NHID), lambda i: (0, 0)),
            pl.BlockSpec((NHID, CP), lambda i: (0, 0)),
            pl.BlockSpec((1, CP), lambda i: (0, 0)),
            pl.BlockSpec((128, NC), lambda i: (i, 0)),
            pl.BlockSpec((1, 1), lambda i: (0, 0)),
        ],
        out_specs=[
            pl.BlockSpec((128, CP), lambda i: (i, 0)),
            pl.BlockSpec((128, CP), lambda i: (i, 0)),
            pl.BlockSpec((128, 16), lambda i: (i, 0)),
        ],
        out_shape=[_f32((NP, CP)), _f32((NP, CP)), _f32((NP, 16))],
    )(x_p, W1, b1r, W2p, b2r, degs_t, t0)


def _make_round(epad):
    chunks = epad // B
    per_tile = chunks // NS
    groups = per_tile // KG

    @functools.partial(
        pl.kernel,
        out_type=(_f32((NP, CP)), _f32((NP, CP))),
        mesh=_MESH,
        compiler_params=pltpu.CompilerParams(use_tc_tiling_on_sc=False),
        scratch_types=[
            pltpu.VMEM_SHARED((NROWS, CP), jnp.float32),
            [pltpu.VMEM((B,), jnp.int32) for _ in range(KG)],
            [pltpu.VMEM((B,), jnp.int32) for _ in range(KG)],
            [pltpu.VMEM((B,), jnp.int32) for _ in range(KG)],
            [pltpu.VMEM((B, CP), jnp.float32) for _ in range(KG)],
            pltpu.VMEM((RW, CP), jnp.float32),
            pltpu.VMEM((RW, CP), jnp.float32),
            pltpu.VMEM((RW, CP), jnp.float32),
            pltpu.VMEM((RW, CP), jnp.float32),
            pltpu.VMEM((RW, 16), jnp.float32),
            pltpu.VMEM((16,), jnp.float32),
            pltpu.SemaphoreType.DMA,
            pltpu.SemaphoreType.DMA,
            pltpu.SemaphoreType.DMA,
        ],
    )
    def rnd(row_hbm, col_hbm, gin_hbm, hin_hbm, dinv_hbm, tk_hbm,
            gout_hbm, hout_hbm,
            acc_sh, rowb, colb, clb, gbuf, zbuf, abuf, obuf, hbuf,
            dbuf, tkb, sem_l, sem_g, sem_s):
        cid = lax.axis_index("c")
        tid = lax.axis_index("s")
        sc_base = cid * HALF
        trash0 = HALF + tid * 16

        pltpu.sync_copy(tk_hbm, tkb)

        @pl.loop(0, RW)
        def _z(i):
            for j in range(CP // 16):
                zbuf[i, pl.ds(j * 16, 16)] = jnp.zeros((16,), jnp.float32)

        @pl.loop(0, NRW)
        def _za(i):
            pltpu.sync_copy(
                zbuf, acc_sh.at[pl.ds(tid * PT_ROWS + i * RW, RW)])

        plsc.subcore_barrier()

        base = tid * per_tile
        iota16 = lax.iota(jnp.int32, 16)

        @pl.loop(0, groups)
        def _grp(gi):
            c0 = base + gi * KG
            dls = []
            for b in range(KG):
                off = (c0 + b) * B
                dls.append(pltpu.async_copy(
                    row_hbm.at[pl.ds(off, B)], rowb[b], sem_l))
                dls.append(pltpu.async_copy(
                    col_hbm.at[pl.ds(off, B)], colb[b], sem_l))
            for d in dls:
                d.wait()
            dgs = []
            for b in range(KG):
                for s in range(B // 16):
                    cv = colb[b][pl.ds(s * 16, 16)] - sc_base
                    oob = (cv < 0) | (cv >= HALF)
                    clb[b][pl.ds(s * 16, 16)] = jnp.where(
                        oob, trash0 + iota16, cv)
                dgs.append(pltpu.async_copy(
                    gin_hbm.at[rowb[b]], gbuf[b], sem_g))
            for d in dgs:
                d.wait()
            dss = [
                pltpu.async_copy(gbuf[b], acc_sh.at[clb[b]], sem_s, add=True)
                for b in range(KG)
            ]
            for d in dss:
                d.wait()

        plsc.subcore_barrier()

        tkv = tkb[...]

        @pl.loop(0, NRW)
        def _dense(ci):
            lrow = tid * PT_ROWS + ci * RW
            grow = sc_base + lrow
            pltpu.sync_copy(acc_sh.at[pl.ds(lrow, RW)], abuf)
            pltpu.sync_copy(gin_hbm.at[pl.ds(grow, RW)], obuf)
            pltpu.sync_copy(hin_hbm.at[pl.ds(grow, RW)], hbuf)
            pltpu.sync_copy(dinv_hbm.at[pl.ds(grow, RW)], dbuf)

            @pl.loop(0, RW)
            def _row(r):
                dv = dbuf[r, :]
                dsq = dv * dv
                ck = tkv * dv
                for j in range(CP // 16):
                    sl = pl.ds(j * 16, 16)
                    t = abuf[r, sl] + obuf[r, sl]
                    abuf[r, sl] = dsq * t
                    hbuf[r, sl] = hbuf[r, sl] + ck * t

            pltpu.sync_copy(abuf, gout_hbm.at[pl.ds(grow, RW)])
            pltpu.sync_copy(hbuf, hout_hbm.at[pl.ds(grow, RW)])

    return rnd


def kernel(x, edge_index, W1, b1, W2, b2, temp):
    E = edge_index.shape[1]
    epad = ((E + 16383) // 16384) * 16384
    npad_e = epad - E

    row = edge_index[0]
    col = edge_index[1]
    pad_idx = (N + (jnp.arange(npad_e, dtype=jnp.int32) % (NP - N))).astype(
        jnp.int32)
    row_p = jnp.concatenate([row, pad_idx])
    col_p = jnp.concatenate([col, pad_idx])

    x_p = jnp.zeros((NP, NFEAT), jnp.float32).at[:N].set(x)
    W2p = jnp.zeros((NHID, CP), jnp.float32).at[:, :C].set(W2)
    b2r = jnp.zeros((1, CP), jnp.float32).at[0, :C].set(b2)
    b1r = b1.reshape(1, NHID)
    t0 = temp[0].reshape(1, 1)

    degs = _make_hist(epad)(col_p)
    degs_t = degs.reshape(NC, NP).T

    g, hid, dinv = _tc_mlp(x_p, W1, b1r, W2p, b2r, degs_t, t0)

    rnd = _make_round(epad)
    for k in range(K):
        tkv = jnp.broadcast_to(temp[k + 1], (16,)).astype(jnp.float32)
        g, hid = rnd(row_p, col_p, g, hid, dinv, tkv)

    return hid[:N, :C]
